# Initial kernel scaffold; baseline (speedup 1.0000x reference)
import jax
import jax.numpy as jnp
from jax.experimental import pallas as pl


def kernel(x, edge_index, edge_attr, global_features, batch, edge_W, edge_b, node_W, node_b, global_W, global_b, edge_dense_W, edge_dense_b, node_dense_W, node_dense_b, global_dense_W, global_dense_b, s2sn_Wih, s2sn_Whh, s2sn_bih, s2sn_bhh, s2se_Wih, s2se_Whh, s2se_bih, s2se_bhh, dense1_W, dense1_b, dense2_W, dense2_b, out_W, out_b):
    raise NotImplementedError("write your pallas kernel here")


# speedup vs baseline: 13.7657x; 13.7657x over previous
import functools
import jax
import jax.numpy as jnp
from jax import lax
from jax.experimental import pallas as pl
from jax.experimental.pallas import tpu as pltpu
from jax.experimental.pallas import tpu_sc as plsc

N = 10000
E = 160000
G = 16
DN = 128
DE = 16
DG = 32
ACCR = 10240


def _table_k(x_ref, batch_ref, gg_ref, ws_ref, wd_ref, out_ref):
    x = x_ref[...]
    oh = (batch_ref[...] == lax.broadcasted_iota(jnp.int32, (1, G), 1)).astype(jnp.float32)
    a = jnp.dot(x, ws_ref[...].T, preferred_element_type=jnp.float32)
    a = a + jnp.dot(oh, gg_ref[...], preferred_element_type=jnp.float32)
    b = jnp.dot(x, wd_ref[...].T, preferred_element_type=jnp.float32)
    pad = jnp.zeros((a.shape[0], 48), jnp.float32)
    out_ref[...] = jnp.concatenate([a, b, oh, pad], axis=1)


def _gg_k(gf_ref, wg_ref, out_ref):
    out_ref[...] = jnp.dot(gf_ref[...], wg_ref[...].T, preferred_element_type=jnp.float32)


def _sc_gather(table, idx):
    info = plsc.get_sparse_core_info()
    nc, ns = info.num_cores, info.num_subcores
    nw = nc * ns
    total = idx.shape[0]
    per_w = total // nw
    chunk = 400
    iters = per_w // chunk
    mesh = plsc.VectorSubcoreMesh(core_axis_name="c", subcore_axis_name="s")

    @functools.partial(
        pl.kernel, mesh=mesh,
        out_type=jax.ShapeDtypeStruct((total, 128), jnp.float32),
        scratch_types=[
            pltpu.VMEM((chunk,), jnp.int32),
            pltpu.VMEM((chunk, 128), jnp.float32),
            pltpu.SemaphoreType.DMA,
        ],
    )
    def k(table_hbm, idx_hbm, out_hbm, idx_v, rows_v, sem):
        wid = lax.axis_index("s") * nc + lax.axis_index("c")
        for i in range(iters):
            base = wid * per_w + i * chunk
            pltpu.sync_copy(idx_hbm.at[pl.ds(base, chunk)], idx_v)
            pltpu.async_copy(table_hbm.at[idx_v], rows_v, sem).wait()
            pltpu.sync_copy(rows_v, out_hbm.at[pl.ds(base, chunk)])

    return k(table, idx)


def _edge_k(gs_ref, gd_ref, ea_ref, we_ref, eb_ref, edw_ref, edb_ref,
            e2f_ref, e2b_ref, enew_ref, ohs_ref, acc_ref):
    gs = gs_ref[0]
    gd = gd_ref[0]
    ea = ea_ref[...]
    ew = jnp.dot(ea, we_ref[...].T, preferred_element_type=jnp.float32) + eb_ref[...]
    h_f = gs[:, :32] + gd[:, 32:64] + ew
    h_b = gd[:, :32] + gs[:, 32:64] + ew
    edw = edw_ref[...].T
    e2f = jnp.dot(h_f, edw, preferred_element_type=jnp.float32) + edb_ref[...]
    e2b = jnp.dot(h_b, edw, preferred_element_type=jnp.float32) + edb_ref[...]
    oh_s = gs[:, 64:80]
    oh_d = gd[:, 64:80]
    c = e2f.shape[0]
    one_col = jnp.ones((c, 1), jnp.float32)
    pad = jnp.zeros((c, 111), jnp.float32)
    e2f_ref[...] = jnp.concatenate([e2f, one_col, pad], axis=1)
    e2b_ref[...] = jnp.concatenate([e2b, one_col, pad], axis=1)
    enew_ref[...] = (e2f + e2b) * 0.5 + ea
    ohs_ref[...] = oh_s

    @pl.when(pl.program_id(0) == 0)
    def _():
        acc_ref[...] = jnp.zeros_like(acc_ref)

    esum = (jnp.dot(oh_s.T, e2f, preferred_element_type=jnp.float32)
            + jnp.dot(oh_d.T, e2b, preferred_element_type=jnp.float32))
    cnt = jnp.sum(oh_s + oh_d, axis=0, keepdims=True)
    acc_ref[0:16, :] += esum
    acc_ref[16:17, :] += cnt


def _sc_scatter(e2xf, e2xb, dst0, src0, zeros):
    info = plsc.get_sparse_core_info()
    nc, ns = info.num_cores, info.num_subcores
    nw = nc * ns
    per_w = E // nw
    chunk = 200
    iters = per_w // chunk
    zrows = ACCR // ns
    mesh = plsc.VectorSubcoreMesh(core_axis_name="c", subcore_axis_name="s")

    @functools.partial(
        pl.kernel, mesh=mesh,
        out_type=jax.ShapeDtypeStruct((2 * ACCR, 128), jnp.float32),
        scratch_types=[
            pltpu.VMEM((chunk,), jnp.int32),
            pltpu.VMEM((chunk, 128), jnp.float32),
            pltpu.VMEM_SHARED((ACCR, 128), jnp.float32),
            pltpu.SemaphoreType.DMA,
        ],
    )
    def k(df_hbm, db_hbm, if_hbm, ib_hbm, z_hbm, out_hbm, idx_v, data_v, acc, sem):
        cid = lax.axis_index("c")
        sid = lax.axis_index("s")
        wid = sid * nc + cid
        pltpu.sync_copy(z_hbm.at[pl.ds(sid * zrows, zrows)], acc.at[pl.ds(sid * zrows, zrows)])
        plsc.subcore_barrier()
        for data_hbm, i_hbm in ((df_hbm, if_hbm), (db_hbm, ib_hbm)):
            for i in range(iters):
                base = wid * per_w + i * chunk
                pltpu.sync_copy(i_hbm.at[pl.ds(base, chunk)], idx_v)
                pltpu.sync_copy(data_hbm.at[pl.ds(base, chunk)], data_v)
                pltpu.sync_copy(data_v, acc.at[idx_v], add=True)
        plsc.subcore_barrier()
        pltpu.sync_copy(acc.at[pl.ds(sid * zrows, zrows)],
                        out_hbm.at[pl.ds(cid * ACCR + sid * zrows, zrows)])

    return k(e2xf, e2xb, dst0, src0, zeros)


def _lstm_gates(qs, h, wih_t, whh_t, bih, bhh):
    gates = (jnp.dot(qs, wih_t, preferred_element_type=jnp.float32) + bih
             + jnp.dot(h, whh_t, preferred_element_type=jnp.float32) + bhh)
    i_g, f_g, g_g, o_g = jnp.split(gates, 4, axis=-1)
    return jax.nn.sigmoid(i_g), jax.nn.sigmoid(f_g), jnp.tanh(g_g), jax.nn.sigmoid(o_g)


def _edge_s2s_k(en_ref, oh_ref, wih_ref, whh_ref, bih_ref, bhh_ref, out_ref,
                h_ref, c_ref, qs_ref, m_ref, s_ref, r_ref):
    step = pl.program_id(0)
    chunk = pl.program_id(1)
    nch = pl.num_programs(1)
    d = DE
    neg = jnp.float32(-1e30)

    @pl.when(chunk == 0)
    def _start_step():
        @pl.when(step == 0)
        def _init():
            h_ref[...] = jnp.zeros_like(h_ref)
            c_ref[...] = jnp.zeros_like(c_ref)
            qs_ref[...] = jnp.zeros_like(qs_ref)

        @pl.when(step > 0)
        def _carry():
            r_fin = r_ref[...] / jnp.clip(s_ref[...].T, 1e-16, None)
            qs_ref[...] = jnp.concatenate([h_ref[...], r_fin], axis=-1)

        i_g, f_g, g_g, o_g = _lstm_gates(qs_ref[...], h_ref[...],
                                         wih_ref[...].T, whh_ref[...].T,
                                         bih_ref[...], bhh_ref[...])
        c_new = f_g * c_ref[...] + i_g * g_g
        c_ref[...] = c_new
        h_ref[...] = o_g * jnp.tanh(c_new)
        m_ref[...] = jnp.full_like(m_ref, neg)
        s_ref[...] = jnp.zeros_like(s_ref)
        r_ref[...] = jnp.zeros_like(r_ref)

    xe = en_ref[...]
    oh = oh_ref[...]
    q = h_ref[...]
    e = jnp.sum(xe * jnp.dot(oh, q, preferred_element_type=jnp.float32),
                axis=-1, keepdims=True)
    ml = jnp.max(jnp.where(oh > 0, e, neg), axis=0, keepdims=True)
    m_old = m_ref[...]
    m_new = jnp.maximum(m_old, ml)
    scale = jnp.exp(m_old - m_new)
    ex = jnp.exp(e - jnp.dot(oh, m_new.T, preferred_element_type=jnp.float32))
    s_ref[...] = s_ref[...] * scale + jnp.dot(ex.T, oh, preferred_element_type=jnp.float32)
    r_ref[...] = (r_ref[...] * scale.T
                  + jnp.dot(oh.T, ex * xe, preferred_element_type=jnp.float32))
    m_ref[...] = m_new

    @pl.when((step == 2) & (chunk == nch - 1))
    def _finish():
        r_fin = r_ref[...] / jnp.clip(s_ref[...].T, 1e-16, None)
        out_ref[...] = jnp.concatenate([h_ref[...], r_fin], axis=-1)


def _final_k(x_ref, scat_ref, batch_ref, gf_ref, eacc_ref, ev_ref,
             wnx_ref, wnem_ref, wng_ref, nb_ref, ndw_ref, ndb_ref,
             wge_ref, wgn_ref, wgg_ref, gb_ref, gdw_ref, gdb_ref,
             wih_ref, whh_ref, bih_ref, bhh_ref,
             d1n_ref, d1e_ref, d1u_ref, d1b_ref, d2w_ref, d2b_ref,
             ow_ref, ob_ref, out_ref):
    x = x_ref[...]
    scat = scat_ref[0:ACCR, :] + scat_ref[ACCR:2 * ACCR, :]
    sums = scat[0:N, 0:16]
    cnt = scat[0:N, 16:17]
    em = sums / jnp.clip(cnt, 1.0, None)
    oh = (batch_ref[...] == lax.broadcasted_iota(jnp.int32, (1, G), 1)).astype(jnp.float32)
    gf = gf_ref[...]
    gvec = jnp.dot(oh, gf, preferred_element_type=jnp.float32)
    h = (jnp.dot(x, wnx_ref[...].T, preferred_element_type=jnp.float32)
         + jnp.dot(em, wnem_ref[...].T, preferred_element_type=jnp.float32)
         + jnp.dot(gvec, wng_ref[...].T, preferred_element_type=jnp.float32)
         + nb_ref[...])
    x_pre = jnp.dot(h, ndw_ref[...].T, preferred_element_type=jnp.float32) + ndb_ref[...]
    ncnt = jnp.sum(oh, axis=0)[:, None]
    n_mean = jnp.dot(oh.T, x_pre, preferred_element_type=jnp.float32) / jnp.clip(ncnt, 1.0, None)
    e_sum = eacc_ref[0:16, :]
    e_cnt = eacc_ref[16:17, :].T
    e_mean = e_sum / jnp.clip(e_cnt, 1.0, None)
    hg = (jnp.dot(e_mean, wge_ref[...].T, preferred_element_type=jnp.float32)
          + jnp.dot(n_mean, wgn_ref[...].T, preferred_element_type=jnp.float32)
          + jnp.dot(gf, wgg_ref[...].T, preferred_element_type=jnp.float32)
          + gb_ref[...])
    u_new = jnp.dot(hg, gdw_ref[...].T, preferred_element_type=jnp.float32) + gdb_ref[...] + gf
    x_new = x_pre + x
    wih = wih_ref[...].T
    whh = whh_ref[...].T
    d = DN
    hh = jnp.zeros((G, d), jnp.float32)
    cc = jnp.zeros((G, d), jnp.float32)
    q_star = jnp.zeros((G, 2 * d), jnp.float32)
    neg = jnp.float32(-1e30)
    for _ in range(3):
        gates = (jnp.dot(q_star, wih, preferred_element_type=jnp.float32) + bih_ref[...]
                 + jnp.dot(hh, whh, preferred_element_type=jnp.float32) + bhh_ref[...])
        i_g, f_g, g_g, o_g = jnp.split(gates, 4, axis=-1)
        i_g = jax.nn.sigmoid(i_g)
        f_g = jax.nn.sigmoid(f_g)
        g_g = jnp.tanh(g_g)
        o_g = jax.nn.sigmoid(o_g)
        cc = f_g * cc + i_g * g_g
        hh = o_g * jnp.tanh(cc)
        q = hh
        e = jnp.sum(x_new * jnp.dot(oh, q, preferred_element_type=jnp.float32), axis=-1)
        m = jnp.max(jnp.where(oh > 0, e[:, None], neg), axis=0)
        m = jnp.where(m > neg * 0.5, m, 0.0)
        ex = jnp.exp(e - jnp.dot(oh, m[:, None], preferred_element_type=jnp.float32)[:, 0])
        s = jnp.dot(oh.T, ex[:, None], preferred_element_type=jnp.float32)[:, 0]
        a = ex / jnp.dot(oh, jnp.clip(s, 1e-16, None)[:, None],
                         preferred_element_type=jnp.float32)[:, 0]
        r = jnp.dot(oh.T, a[:, None] * x_new, preferred_element_type=jnp.float32)
        q_star = jnp.concatenate([q, r], axis=-1)
    nv = q_star
    o = (jnp.dot(nv, d1n_ref[...].T, preferred_element_type=jnp.float32)
         + jnp.dot(ev_ref[...], d1e_ref[...].T, preferred_element_type=jnp.float32)
         + jnp.dot(u_new, d1u_ref[...].T, preferred_element_type=jnp.float32)
         + d1b_ref[...])
    o = jnp.dot(o, d2w_ref[...].T, preferred_element_type=jnp.float32) + d2b_ref[...]
    out_ref[...] = jnp.sum(o * ow_ref[...], axis=1, keepdims=True) + ob_ref[...]


def kernel(x, edge_index, edge_attr, global_features, batch, edge_W, edge_b,
           node_W, node_b, global_W, global_b, edge_dense_W, edge_dense_b,
           node_dense_W, node_dense_b, global_dense_W, global_dense_b,
           s2sn_Wih, s2sn_Whh, s2sn_bih, s2sn_bhh,
           s2se_Wih, s2se_Whh, s2se_bih, s2se_bhh,
           dense1_W, dense1_b, dense2_W, dense2_b, out_W, out_b):
    src0 = edge_index[0]
    dst0 = edge_index[1]
    batch2 = batch.reshape(N, 1)

    ws = edge_W[:, 0:DN]
    wd = edge_W[:, DN:2 * DN]
    we = edge_W[:, 2 * DN:2 * DN + DE]
    wg = edge_W[:, 2 * DN + DE:]
    gg = pl.pallas_call(
        _gg_k, out_shape=jax.ShapeDtypeStruct((G, 32), jnp.float32))(global_features, wg)

    table = pl.pallas_call(
        _table_k, out_shape=jax.ShapeDtypeStruct((N, 128), jnp.float32))(
            x, batch2, gg, ws, wd)

    idx_all = jnp.concatenate([src0, dst0])
    gth = _sc_gather(table, idx_all)
    g3 = gth.reshape(2, E, 128)

    CE = 2000
    steps = E // CE
    full = lambda shp: pl.BlockSpec(shp, lambda i: tuple(0 for _ in shp))
    e2xf, e2xb, e_new, oh_s, eacc = pl.pallas_call(
        _edge_k,
        grid=(steps,),
        in_specs=[
            pl.BlockSpec((1, CE, 128), lambda i: (0, i, 0)),
            pl.BlockSpec((1, CE, 128), lambda i: (1, i, 0)),
            pl.BlockSpec((CE, DE), lambda i: (i, 0)),
            full((32, DE)),
            full((1, 32)),
            full((DE, 32)),
            full((1, DE)),
        ],
        out_specs=[
            pl.BlockSpec((CE, 128), lambda i: (i, 0)),
            pl.BlockSpec((CE, 128), lambda i: (i, 0)),
            pl.BlockSpec((CE, DE), lambda i: (i, 0)),
            pl.BlockSpec((CE, 16), lambda i: (i, 0)),
            pl.BlockSpec((24, 16), lambda i: (0, 0)),
        ],
        out_shape=[
            jax.ShapeDtypeStruct((E, 128), jnp.float32),
            jax.ShapeDtypeStruct((E, 128), jnp.float32),
            jax.ShapeDtypeStruct((E, DE), jnp.float32),
            jax.ShapeDtypeStruct((E, 16), jnp.float32),
            jax.ShapeDtypeStruct((24, 16), jnp.float32),
        ],
    )(g3, g3, edge_attr, we, edge_b.reshape(1, 32), edge_dense_W,
      edge_dense_b.reshape(1, DE))

    zeros = jnp.zeros((ACCR, 128), jnp.float32)
    scat = _sc_scatter(e2xf, e2xb, dst0, src0, zeros)

    CS = 4000
    nch = E // CS
    ev = pl.pallas_call(
        _edge_s2s_k,
        grid=(3, nch),
        in_specs=[
            pl.BlockSpec((CS, DE), lambda s, c: (c, 0)),
            pl.BlockSpec((CS, 16), lambda s, c: (c, 0)),
            pl.BlockSpec((4 * DE, 2 * DE), lambda s, c: (0, 0)),
            pl.BlockSpec((4 * DE, DE), lambda s, c: (0, 0)),
            pl.BlockSpec((1, 4 * DE), lambda s, c: (0, 0)),
            pl.BlockSpec((1, 4 * DE), lambda s, c: (0, 0)),
        ],
        out_specs=pl.BlockSpec((G, 2 * DE), lambda s, c: (0, 0)),
        out_shape=jax.ShapeDtypeStruct((G, 2 * DE), jnp.float32),
        scratch_shapes=[
            pltpu.VMEM((G, DE), jnp.float32),
            pltpu.VMEM((G, DE), jnp.float32),
            pltpu.VMEM((G, 2 * DE), jnp.float32),
            pltpu.VMEM((1, G), jnp.float32),
            pltpu.VMEM((1, G), jnp.float32),
            pltpu.VMEM((G, DE), jnp.float32),
        ],
    )(e_new, oh_s, s2se_Wih, s2se_Whh,
      s2se_bih.reshape(1, 4 * DE), s2se_bhh.reshape(1, 4 * DE))

    wnx = node_W[:, 0:DN]
    wnem = node_W[:, DN:DN + DE]
    wng = node_W[:, DN + DE:]
    wge = global_W[:, 0:DE]
    wgn = global_W[:, DE:DE + DN]
    wgg = global_W[:, DE + DN:]
    d1n = dense1_W[:, 0:2 * DN]
    d1e = dense1_W[:, 2 * DN:2 * DN + 2 * DE]
    d1u = dense1_W[:, 2 * DN + 2 * DE:]

    out = pl.pallas_call(
        _final_k, out_shape=jax.ShapeDtypeStruct((G, 1), jnp.float32))(
            x, scat, batch2, global_features, eacc, ev,
            wnx, wnem, wng, node_b.reshape(1, 32), node_dense_W,
            node_dense_b.reshape(1, DN),
            wge, wgn, wgg, global_b.reshape(1, 32), global_dense_W,
            global_dense_b.reshape(1, DG),
            s2sn_Wih, s2sn_Whh, s2sn_bih.reshape(1, 4 * DN), s2sn_bhh.reshape(1, 4 * DN),
            d1n, d1e, d1u, dense1_b.reshape(1, 32), dense2_W, dense2_b.reshape(1, 16),
            out_W, jnp.broadcast_to(out_b.reshape(1, 1), (G, 1)))
    return out
